# baseline (speedup 1.0000x reference)
import functools

import jax
import jax.numpy as jnp
from jax import lax
from jax.experimental import pallas as pl
from jax.experimental.pallas import tpu as pltpu
from jax.experimental.pallas import tpu_sc as plsc

_N_SEG = 100000

_GATHER_DNUMS = lax.GatherDimensionNumbers(
    offset_dims=(), collapsed_slice_dims=(0,), start_index_map=(0,))


def _vgather(v, idx):
    return lax.gather(v, idx[:, None], _GATHER_DNUMS, (1,),
                      mode=lax.GatherScatterMode.PROMISE_IN_BOUNDS)


def _make_sc_kernel(n_out_pad, n_edges, chunk, win, num_cores, num_subcores):
    n_workers = num_cores * num_subcores
    seg_per_w = n_out_pad // n_workers
    mesh = plsc.VectorSubcoreMesh(core_axis_name="c", subcore_axis_name="s")
    vpc = chunk // 16

    @functools.partial(
        pl.kernel,
        out_type=jax.ShapeDtypeStruct((n_out_pad,), jnp.float32),
        mesh=mesh,
        scratch_types=[
            pltpu.VMEM((100000,), jnp.float32),
            pltpu.VMEM((win,), jnp.float32),
            pltpu.VMEM((2 * chunk,), jnp.int32),
            pltpu.VMEM((2 * chunk,), jnp.int32),
            pltpu.VMEM((96,), jnp.int32),
            pltpu.VMEM((16,), jnp.int32),
            pltpu.VMEM((16,), jnp.float32),
            pltpu.SemaphoreType.DMA((2,)),
        ],
        compiler_params=pltpu.CompilerParams(needs_layout_passes=False),
    )
    def run(x_hbm, ptrs_hbm, csr_hbm, out_hbm,
            x_v, win_v, ptrs_v, csr_v, scan_v, seg_sc, acc_sc, lsem):
        c = lax.axis_index("c")
        s = lax.axis_index("s")
        w = s * num_cores + c
        wbase = w * seg_per_w

        iota16 = lax.broadcasted_iota(jnp.int32, (16,), 0)
        sh_idx = [jnp.maximum(iota16 - k, 0) for k in (1, 2, 4, 8)]
        sh_msk = [iota16 >= k for k in (1, 2, 4, 8)]
        nxt_idx = jnp.minimum(iota16 + 1, 15)

        pltpu.sync_copy(x_hbm, x_v)

        zvec = jnp.zeros((16,), jnp.float32)

        def zbody(i, carry):
            win_v[pl.ds(i * 16, 16)] = zvec
            return carry

        lax.fori_loop(0, win // 16, zbody, 0)

        def lower_bound(t):
            def body(i, state):
                lo, hi = state
                mid = ((lo + hi) // 2) // 16 * 16
                mid = jnp.minimum(mid, n_edges - 16)
                pltpu.sync_copy(csr_hbm.at[pl.ds(mid, 16)],
                                scan_v.at[pl.ds(0, 16)])
                cmid = scan_v[pl.ds(0, 16)][0]
                small = hi - lo <= 64
                ge = cmid >= t
                new_lo = jnp.where(small, lo, jnp.where(ge, lo, mid + 1))
                new_hi = jnp.where(small, hi, jnp.where(ge, mid, hi))
                return (new_lo, new_hi)

            lo, hi = lax.fori_loop(0, 24, body, (jnp.int32(0),
                                                 jnp.int32(n_edges)))
            base = jnp.minimum(lo // 16 * 16, n_edges - 96)
            pltpu.sync_copy(csr_hbm.at[pl.ds(base, 96)], scan_v)

            cnt = jnp.int32(0)
            for k in range(6):
                v = scan_v[pl.ds(k * 16, 16)]
                for i in range(16):
                    cnt = cnt + jnp.where(v[i] < t, 1, 0)
            return base + cnt

        e_lo = lower_bound(wbase)
        e_hi = lower_bound(wbase + seg_per_w)

        a_lo = e_lo // 8 * 8
        nfull = (e_hi - a_lo) // chunk
        tail_proc = a_lo + nfull * chunk
        tail_base = jnp.minimum(tail_proc, n_edges - chunk)

        def start_load(base, par):
            boff = par * chunk
            pltpu.async_copy(ptrs_hbm.at[pl.ds(base, chunk)],
                             ptrs_v.at[pl.ds(boff, chunk)], lsem.at[par])
            pltpu.async_copy(csr_hbm.at[pl.ds(base, chunk)],
                             csr_v.at[pl.ds(boff, chunk)], lsem.at[par])

        def wait_load(base, par):
            boff = par * chunk
            pltpu.make_async_copy(ptrs_hbm.at[pl.ds(base, chunk)],
                                  ptrs_v.at[pl.ds(boff, chunk)],
                                  lsem.at[par]).wait()
            pltpu.make_async_copy(csr_hbm.at[pl.ds(base, chunk)],
                                  csr_v.at[pl.ds(boff, chunk)],
                                  lsem.at[par]).wait()

        def compute(base, par, lob, hib):
            boff = par * chunk

            def vbody(j, carry):
                off = boff + j * 16
                seg = csr_v[pl.ds(off, 16)]
                idx = ptrs_v[pl.ds(off, 16)]
                vals = plsc.load_gather(x_v, [idx])
                glob = (base + j * 16) + iota16
                inr = (glob >= lob) & (glob < hib)
                vals = jnp.where(inr, vals, jnp.float32(0.0))
                acc = vals
                seg_sc[...] = seg
                for k in range(4):
                    seg_k = plsc.load_gather(seg_sc, [sh_idx[k]])
                    acc_sc[...] = acc
                    acc_k = plsc.load_gather(acc_sc, [sh_idx[k]])
                    take = (seg == seg_k) & sh_msk[k]
                    acc = acc + jnp.where(take, acc_k, jnp.float32(0.0))
                seg_n = plsc.load_gather(seg_sc, [nxt_idx])
                is_last = (seg != seg_n) | (iota16 == 15)
                wmask = is_last & inr
                offs = jnp.clip(seg - wbase, 0, win - 1)
                plsc.addupdate_scatter(win_v, [offs], acc, mask=wmask)
                return carry

            lax.fori_loop(0, vpc, vbody, 0)

        first_base = jnp.where(nfull > 0, a_lo, tail_base)
        start_load(first_base, 0)

        def mbody(i, carry):
            par = lax.rem(i, 2)
            base = a_lo + i * chunk
            wait_load(base, par)
            nbase = jnp.where(i + 1 < nfull, a_lo + (i + 1) * chunk,
                              tail_base)
            start_load(nbase, 1 - par)
            compute(base, par, e_lo, e_hi)
            return carry

        lax.fori_loop(0, nfull, mbody, 0)

        par_t = lax.rem(nfull, 2)
        wait_load(tail_base, par_t)
        compute(tail_base, par_t, jnp.maximum(tail_proc, e_lo), e_hi)

        pltpu.sync_copy(win_v.at[pl.ds(0, seg_per_w)],
                        out_hbm.at[pl.ds(wbase, seg_per_w)])

    return run


def kernel(x, ptrs, csr):
    n_edges = ptrs.shape[0]
    info = plsc.get_sparse_core_info()
    num_cores, num_subcores = info.num_cores, info.num_subcores
    n_workers = num_cores * num_subcores

    align = n_workers * 8
    n_out_pad = ((_N_SEG + align - 1) // align) * align
    chunk = 4992
    win = 3136

    run = _make_sc_kernel(n_out_pad, n_edges, chunk, win,
                          num_cores, num_subcores)
    out = run(x, ptrs, csr)
    return out[:_N_SEG]


# speedup vs baseline: 1.8212x; 1.8212x over previous
import functools

import jax
import jax.numpy as jnp
from jax import lax
from jax.experimental import pallas as pl
from jax.experimental.pallas import tpu as pltpu
from jax.experimental.pallas import tpu_sc as plsc

_N_SEG = 100000

def _make_sc_kernel(n_out_pad, n_edges, chunk, win, num_cores, num_subcores):
    n_workers = num_cores * num_subcores
    seg_per_w = n_out_pad // n_workers
    mesh = plsc.VectorSubcoreMesh(core_axis_name="c", subcore_axis_name="s")
    vpc = chunk // 16

    @functools.partial(
        pl.kernel,
        out_type=jax.ShapeDtypeStruct((n_out_pad,), jnp.float32),
        mesh=mesh,
        scratch_types=[
            pltpu.VMEM((100000,), jnp.float32),
            pltpu.VMEM((win,), jnp.float32),
            pltpu.VMEM((2 * chunk,), jnp.int32),
            pltpu.VMEM((2 * chunk + 16,), jnp.int32),
            pltpu.VMEM((96,), jnp.int32),
            pltpu.SemaphoreType.DMA((2,)),
        ],
        compiler_params=pltpu.CompilerParams(needs_layout_passes=False),
    )
    def run(x_hbm, ptrs_hbm, csr_hbm, out_hbm,
            x_v, win_v, ptrs_v, csr_v, scan_v, lsem):
        c = lax.axis_index("c")
        s = lax.axis_index("s")
        w = s * num_cores + c
        wbase = w * seg_per_w

        iota16 = lax.broadcasted_iota(jnp.int32, (16,), 0)
        not_last_lane = iota16 < 15

        pltpu.sync_copy(x_hbm, x_v)

        zvec = jnp.zeros((16,), jnp.float32)

        def zbody(i, carry):
            win_v[pl.ds(i * 16, 16)] = zvec
            return carry

        lax.fori_loop(0, win // 16, zbody, 0)

        def lower_bound(t):
            def body(i, state):
                lo, hi = state
                mid = ((lo + hi) // 2) // 16 * 16
                mid = jnp.minimum(mid, n_edges - 16)
                pltpu.sync_copy(csr_hbm.at[pl.ds(mid, 16)],
                                scan_v.at[pl.ds(0, 16)])
                cmid = scan_v[pl.ds(0, 16)][0]
                small = hi - lo <= 64
                ge = cmid >= t
                new_lo = jnp.where(small, lo, jnp.where(ge, lo, mid + 1))
                new_hi = jnp.where(small, hi, jnp.where(ge, mid, hi))
                return (new_lo, new_hi)

            lo, hi = lax.fori_loop(0, 24, body, (jnp.int32(0),
                                                 jnp.int32(n_edges)))
            base = jnp.minimum(lo // 16 * 16, n_edges - 96)
            pltpu.sync_copy(csr_hbm.at[pl.ds(base, 96)], scan_v)

            cnt = jnp.int32(0)
            for k in range(6):
                v = scan_v[pl.ds(k * 16, 16)]
                for i in range(16):
                    cnt = cnt + jnp.where(v[i] < t, 1, 0)
            return base + cnt

        e_lo = lower_bound(wbase)
        e_hi = lower_bound(wbase + seg_per_w)

        a_lo = e_lo // 8 * 8
        nfull = (e_hi - a_lo) // chunk
        tail_proc = a_lo + nfull * chunk
        tail_base = jnp.minimum(tail_proc, n_edges - chunk)

        def start_load(base, par):
            boff = par * chunk
            pltpu.async_copy(ptrs_hbm.at[pl.ds(base, chunk)],
                             ptrs_v.at[pl.ds(boff, chunk)], lsem.at[par])
            pltpu.async_copy(csr_hbm.at[pl.ds(base, chunk)],
                             csr_v.at[pl.ds(boff, chunk)], lsem.at[par])

        def wait_load(base, par):
            boff = par * chunk
            pltpu.make_async_copy(ptrs_hbm.at[pl.ds(base, chunk)],
                                  ptrs_v.at[pl.ds(boff, chunk)],
                                  lsem.at[par]).wait()
            pltpu.make_async_copy(csr_hbm.at[pl.ds(base, chunk)],
                                  csr_v.at[pl.ds(boff, chunk)],
                                  lsem.at[par]).wait()

        def compute(base, par, lob, hib):
            boff = par * chunk
            unroll = 4

            def vbody(jj, carry):
                for u in range(unroll):
                    j = jj * unroll + u
                    off = boff + j * 16
                    seg = csr_v[pl.ds(off, 16)]
                    segn = csr_v[pl.ds(off + 1, 16)]
                    idx = ptrs_v[pl.ds(off, 16)]
                    vals = plsc.load_gather(x_v, [idx])
                    glob = (base + j * 16) + iota16
                    inr = (glob >= lob) & (glob < hib)
                    vals = jnp.where(inr, vals, jnp.float32(0.0))
                    p = plsc.cumsum(vals)
                    is_last = (seg != segn) | (iota16 == 15)
                    offs = jnp.clip(seg - wbase, 0, win - 1)
                    offsn = jnp.clip(segn - wbase, 0, win - 1)
                    maska = is_last & inr
                    inrn = (glob >= lob - 1) & (glob < hib - 1)
                    maskb = is_last & inrn & not_last_lane
                    plsc.addupdate_scatter(win_v, [offs], p, mask=maska)
                    plsc.addupdate_scatter(win_v, [offsn], -p, mask=maskb)
                return carry

            lax.fori_loop(0, vpc // unroll, vbody, 0)

        first_base = jnp.where(nfull > 0, a_lo, tail_base)
        start_load(first_base, 0)

        def mbody(i, carry):
            par = lax.rem(i, 2)
            base = a_lo + i * chunk
            wait_load(base, par)
            nbase = jnp.where(i + 1 < nfull, a_lo + (i + 1) * chunk,
                              tail_base)
            start_load(nbase, 1 - par)
            compute(base, par, e_lo, e_hi)
            return carry

        lax.fori_loop(0, nfull, mbody, 0)

        par_t = lax.rem(nfull, 2)
        wait_load(tail_base, par_t)
        compute(tail_base, par_t, jnp.maximum(tail_proc, e_lo), e_hi)

        pltpu.sync_copy(win_v.at[pl.ds(0, seg_per_w)],
                        out_hbm.at[pl.ds(wbase, seg_per_w)])

    return run


def kernel(x, ptrs, csr):
    n_edges = ptrs.shape[0]
    info = plsc.get_sparse_core_info()
    num_cores, num_subcores = info.num_cores, info.num_subcores
    n_workers = num_cores * num_subcores

    align = n_workers * 8
    n_out_pad = ((_N_SEG + align - 1) // align) * align
    chunk = 4992
    win = 3136

    run = _make_sc_kernel(n_out_pad, n_edges, chunk, win,
                          num_cores, num_subcores)
    out = run(x, ptrs, csr)
    return out[:_N_SEG]


# speedup vs baseline: 3.0433x; 1.6710x over previous
import functools

import jax
import jax.numpy as jnp
from jax import lax
from jax.experimental import pallas as pl
from jax.experimental.pallas import tpu as pltpu
from jax.experimental.pallas import tpu_sc as plsc

_N_SEG = 100000

def _make_sc_kernel(n_out_pad, n_edges, chunk, win, num_cores, num_subcores):
    n_workers = num_cores * num_subcores
    seg_per_w = n_out_pad // n_workers
    mesh = plsc.VectorSubcoreMesh(core_axis_name="c", subcore_axis_name="s")
    vpc = chunk // 16

    @functools.partial(
        pl.kernel,
        out_type=jax.ShapeDtypeStruct((n_out_pad,), jnp.float32),
        mesh=mesh,
        scratch_types=[
            pltpu.VMEM((100000,), jnp.float32),
            pltpu.VMEM((win,), jnp.float32),
            pltpu.VMEM((2 * chunk,), jnp.int32),
            pltpu.VMEM((2 * chunk + 16,), jnp.int32),
            pltpu.VMEM((96,), jnp.int32),
            pltpu.SemaphoreType.DMA((2,)),
        ],
        compiler_params=pltpu.CompilerParams(needs_layout_passes=False),
    )
    def run(x_hbm, ptrs_hbm, csr_hbm, out_hbm,
            x_v, win_v, ptrs_v, csr_v, scan_v, lsem):
        c = lax.axis_index("c")
        s = lax.axis_index("s")
        w = s * num_cores + c
        wbase = w * seg_per_w

        iota16 = lax.broadcasted_iota(jnp.int32, (16,), 0)
        not_last_lane = iota16 < 15

        pltpu.sync_copy(x_hbm, x_v)

        zvec = jnp.zeros((16,), jnp.float32)

        def zbody(i, carry):
            win_v[pl.ds(i * 16, 16)] = zvec
            return carry

        lax.fori_loop(0, win // 16, zbody, 0)

        def lower_bound(t):
            def body(i, state):
                lo, hi = state
                mid = ((lo + hi) // 2) // 16 * 16
                mid = jnp.minimum(mid, n_edges - 16)
                pltpu.sync_copy(csr_hbm.at[pl.ds(mid, 16)],
                                scan_v.at[pl.ds(0, 16)])
                cmid = scan_v[pl.ds(0, 16)][0]
                small = hi - lo <= 64
                ge = cmid >= t
                new_lo = jnp.where(small, lo, jnp.where(ge, lo, mid + 1))
                new_hi = jnp.where(small, hi, jnp.where(ge, mid, hi))
                return (new_lo, new_hi)

            lo, hi = lax.fori_loop(0, 24, body, (jnp.int32(0),
                                                 jnp.int32(n_edges)))
            base = jnp.minimum(lo // 16 * 16, n_edges - 96)
            pltpu.sync_copy(csr_hbm.at[pl.ds(base, 96)], scan_v)

            cnt = jnp.int32(0)
            for k in range(6):
                v = scan_v[pl.ds(k * 16, 16)]
                for i in range(16):
                    cnt = cnt + jnp.where(v[i] < t, 1, 0)
            return base + cnt

        e_lo = lower_bound(wbase)
        e_hi = lower_bound(wbase + seg_per_w)

        a_lo = e_lo // 8 * 8
        nfull = (e_hi - a_lo) // chunk
        tail_proc = a_lo + nfull * chunk
        tail_base = jnp.minimum(tail_proc, n_edges - chunk)

        def start_load(base, par):
            boff = par * chunk
            pltpu.async_copy(ptrs_hbm.at[pl.ds(base, chunk)],
                             ptrs_v.at[pl.ds(boff, chunk)], lsem.at[par])
            pltpu.async_copy(csr_hbm.at[pl.ds(base, chunk)],
                             csr_v.at[pl.ds(boff, chunk)], lsem.at[par])

        def wait_load(base, par):
            boff = par * chunk
            pltpu.make_async_copy(ptrs_hbm.at[pl.ds(base, chunk)],
                                  ptrs_v.at[pl.ds(boff, chunk)],
                                  lsem.at[par]).wait()
            pltpu.make_async_copy(csr_hbm.at[pl.ds(base, chunk)],
                                  csr_v.at[pl.ds(boff, chunk)],
                                  lsem.at[par]).wait()

        def compute(base, par, lob, hib):
            boff = par * chunk

            @plsc.parallel_loop(0, vpc, unroll=4)
            def vbody(j):
                off = boff + j * 16
                seg = csr_v[pl.ds(off, 16)]
                segn = csr_v[pl.ds(off + 1, 16)]
                idx = ptrs_v[pl.ds(off, 16)]
                vals = plsc.load_gather(x_v, [idx])
                glob = (base + j * 16) + iota16
                inr = (glob >= lob) & (glob < hib)
                vals = jnp.where(inr, vals, jnp.float32(0.0))
                p = plsc.cumsum(vals)
                is_last = (seg != segn) | (iota16 == 15)
                offs = jnp.clip(seg - wbase, 0, win - 1)
                offsn = jnp.clip(segn - wbase, 0, win - 1)
                maska = is_last & inr
                inrn = (glob >= lob - 1) & (glob < hib - 1)
                maskb = is_last & inrn & not_last_lane
                plsc.addupdate_scatter(win_v, [offs], p, mask=maska)
                plsc.addupdate_scatter(win_v, [offsn], -p, mask=maskb)

        first_base = jnp.where(nfull > 0, a_lo, tail_base)
        start_load(first_base, 0)

        def mbody(i, carry):
            par = lax.rem(i, 2)
            base = a_lo + i * chunk
            wait_load(base, par)
            nbase = jnp.where(i + 1 < nfull, a_lo + (i + 1) * chunk,
                              tail_base)
            start_load(nbase, 1 - par)
            compute(base, par, e_lo, e_hi)
            return carry

        lax.fori_loop(0, nfull, mbody, 0)

        par_t = lax.rem(nfull, 2)
        wait_load(tail_base, par_t)
        compute(tail_base, par_t, jnp.maximum(tail_proc, e_lo), e_hi)

        pltpu.sync_copy(win_v.at[pl.ds(0, seg_per_w)],
                        out_hbm.at[pl.ds(wbase, seg_per_w)])

    return run


def kernel(x, ptrs, csr):
    n_edges = ptrs.shape[0]
    info = plsc.get_sparse_core_info()
    num_cores, num_subcores = info.num_cores, info.num_subcores
    n_workers = num_cores * num_subcores

    align = n_workers * 8
    n_out_pad = ((_N_SEG + align - 1) // align) * align
    chunk = 4992
    win = 3136

    run = _make_sc_kernel(n_out_pad, n_edges, chunk, win,
                          num_cores, num_subcores)
    out = run(x, ptrs, csr)
    return out[:_N_SEG]


# speedup vs baseline: 3.8680x; 1.2710x over previous
import functools

import jax
import jax.numpy as jnp
from jax import lax
from jax.experimental import pallas as pl
from jax.experimental.pallas import tpu as pltpu
from jax.experimental.pallas import tpu_sc as plsc

_N_SEG = 100000

def _make_sc_kernel(n_out_pad, n_edges, chunk, win, num_cores, num_subcores):
    n_workers = num_cores * num_subcores
    seg_per_w = n_out_pad // n_workers
    mesh = plsc.VectorSubcoreMesh(core_axis_name="c", subcore_axis_name="s")
    vpc = chunk // 16

    @functools.partial(
        pl.kernel,
        out_type=jax.ShapeDtypeStruct((n_out_pad,), jnp.float32),
        mesh=mesh,
        scratch_types=[
            pltpu.VMEM((100000,), jnp.float32),
            pltpu.VMEM((win,), jnp.float32),
            pltpu.VMEM((2 * chunk,), jnp.int32),
            pltpu.VMEM((2 * chunk + 16,), jnp.int32),
            pltpu.VMEM((96,), jnp.int32),
            pltpu.SemaphoreType.DMA((2,)),
        ],
        compiler_params=pltpu.CompilerParams(needs_layout_passes=False),
    )
    def run(x_hbm, ptrs_hbm, csr_hbm, out_hbm,
            x_v, win_v, ptrs_v, csr_v, scan_v, lsem):
        c = lax.axis_index("c")
        s = lax.axis_index("s")
        w = s * num_cores + c
        wbase = w * seg_per_w

        iota16 = lax.broadcasted_iota(jnp.int32, (16,), 0)
        not_last_lane = iota16 < 15

        pltpu.sync_copy(x_hbm, x_v)

        zvec = jnp.zeros((16,), jnp.float32)

        def zbody(i, carry):
            win_v[pl.ds(i * 16, 16)] = zvec
            return carry

        lax.fori_loop(0, win // 16, zbody, 0)

        def lower_bound(t):
            def body(i, state):
                lo, hi = state
                mid = ((lo + hi) // 2) // 16 * 16
                mid = jnp.minimum(mid, n_edges - 16)
                pltpu.sync_copy(csr_hbm.at[pl.ds(mid, 16)],
                                scan_v.at[pl.ds(0, 16)])
                cmid = scan_v[pl.ds(0, 16)][0]
                small = hi - lo <= 64
                ge = cmid >= t
                new_lo = jnp.where(small, lo, jnp.where(ge, lo, mid + 1))
                new_hi = jnp.where(small, hi, jnp.where(ge, mid, hi))
                return (new_lo, new_hi)

            lo, hi = lax.fori_loop(0, 24, body, (jnp.int32(0),
                                                 jnp.int32(n_edges)))
            base = jnp.minimum(lo // 16 * 16, n_edges - 96)
            pltpu.sync_copy(csr_hbm.at[pl.ds(base, 96)], scan_v)

            cnt = jnp.int32(0)
            for k in range(6):
                v = scan_v[pl.ds(k * 16, 16)]
                for i in range(16):
                    cnt = cnt + jnp.where(v[i] < t, 1, 0)
            return base + cnt

        e_lo = lower_bound(wbase)
        e_hi = lower_bound(wbase + seg_per_w)

        a_lo = e_lo // 8 * 8
        nfull = (e_hi - a_lo) // chunk
        tail_proc = a_lo + nfull * chunk
        tail_base = jnp.minimum(tail_proc, n_edges - chunk)

        def start_load(base, par):
            boff = par * chunk
            pltpu.async_copy(ptrs_hbm.at[pl.ds(base, chunk)],
                             ptrs_v.at[pl.ds(boff, chunk)], lsem.at[par])
            pltpu.async_copy(csr_hbm.at[pl.ds(base, chunk)],
                             csr_v.at[pl.ds(boff, chunk)], lsem.at[par])

        def wait_load(base, par):
            boff = par * chunk
            pltpu.make_async_copy(ptrs_hbm.at[pl.ds(base, chunk)],
                                  ptrs_v.at[pl.ds(boff, chunk)],
                                  lsem.at[par]).wait()
            pltpu.make_async_copy(csr_hbm.at[pl.ds(base, chunk)],
                                  csr_v.at[pl.ds(boff, chunk)],
                                  lsem.at[par]).wait()

        def compute(base, par, lob, hib):
            boff = par * chunk

            @plsc.parallel_loop(0, vpc, unroll=8)
            def vbody(j):
                off = boff + j * 16
                seg = csr_v[pl.ds(off, 16)]
                segn = csr_v[pl.ds(off + 1, 16)]
                idx = ptrs_v[pl.ds(off, 16)]
                vals = plsc.load_gather(x_v, [idx])
                glob = (base + j * 16) + iota16
                inr = (glob >= lob) & (glob < hib)
                vals = jnp.where(inr, vals, jnp.float32(0.0))
                p = plsc.cumsum(vals)
                is_last = (seg != segn) | (iota16 == 15)
                offs = jnp.clip(seg - wbase, 0, win - 1)
                offsn = jnp.clip(segn - wbase, 0, win - 1)
                maska = is_last & inr
                inrn = (glob >= lob - 1) & (glob < hib - 1)
                maskb = is_last & inrn & not_last_lane
                plsc.addupdate_scatter(win_v, [offs], p, mask=maska)
                plsc.addupdate_scatter(win_v, [offsn], -p, mask=maskb)

        first_base = jnp.where(nfull > 0, a_lo, tail_base)
        start_load(first_base, 0)

        def mbody(i, carry):
            par = lax.rem(i, 2)
            base = a_lo + i * chunk
            wait_load(base, par)
            nbase = jnp.where(i + 1 < nfull, a_lo + (i + 1) * chunk,
                              tail_base)
            start_load(nbase, 1 - par)
            compute(base, par, e_lo, e_hi)
            return carry

        lax.fori_loop(0, nfull, mbody, 0)

        par_t = lax.rem(nfull, 2)
        wait_load(tail_base, par_t)
        compute(tail_base, par_t, jnp.maximum(tail_proc, e_lo), e_hi)

        pltpu.sync_copy(win_v.at[pl.ds(0, seg_per_w)],
                        out_hbm.at[pl.ds(wbase, seg_per_w)])

    return run


def kernel(x, ptrs, csr):
    n_edges = ptrs.shape[0]
    info = plsc.get_sparse_core_info()
    num_cores, num_subcores = info.num_cores, info.num_subcores
    n_workers = num_cores * num_subcores

    align = n_workers * 8
    n_out_pad = ((_N_SEG + align - 1) // align) * align
    chunk = 4992
    win = 3136

    run = _make_sc_kernel(n_out_pad, n_edges, chunk, win,
                          num_cores, num_subcores)
    out = run(x, ptrs, csr)
    return out[:_N_SEG]


# speedup vs baseline: 3.9323x; 1.0166x over previous
import functools

import jax
import jax.numpy as jnp
from jax import lax
from jax.experimental import pallas as pl
from jax.experimental.pallas import tpu as pltpu
from jax.experimental.pallas import tpu_sc as plsc

_N_SEG = 100000

def _make_sc_kernel(n_out_pad, n_edges, chunk, win, num_cores, num_subcores):
    n_workers = num_cores * num_subcores
    seg_per_w = n_out_pad // n_workers
    mesh = plsc.VectorSubcoreMesh(core_axis_name="c", subcore_axis_name="s")
    vpc = chunk // 16

    @functools.partial(
        pl.kernel,
        out_type=jax.ShapeDtypeStruct((n_out_pad,), jnp.float32),
        mesh=mesh,
        scratch_types=[
            pltpu.VMEM((100000,), jnp.float32),
            pltpu.VMEM((win,), jnp.float32),
            pltpu.VMEM((2 * chunk,), jnp.int32),
            pltpu.VMEM((2 * chunk + 16,), jnp.int32),
            pltpu.VMEM((96,), jnp.int32),
            pltpu.SemaphoreType.DMA((2,)),
        ],
        compiler_params=pltpu.CompilerParams(needs_layout_passes=False),
    )
    def run(x_hbm, ptrs_hbm, csr_hbm, out_hbm,
            x_v, win_v, ptrs_v, csr_v, scan_v, lsem):
        c = lax.axis_index("c")
        s = lax.axis_index("s")
        w = s * num_cores + c
        wbase = w * seg_per_w

        iota16 = lax.broadcasted_iota(jnp.int32, (16,), 0)
        not_last_lane = iota16 < 15

        pltpu.sync_copy(x_hbm, x_v)

        zvec = jnp.zeros((16,), jnp.float32)

        def zbody(i, carry):
            win_v[pl.ds(i * 16, 16)] = zvec
            return carry

        lax.fori_loop(0, win // 16, zbody, 0)

        def lower_bound(t):
            def body(i, state):
                lo, hi = state
                mid = ((lo + hi) // 2) // 16 * 16
                mid = jnp.minimum(mid, n_edges - 16)
                pltpu.sync_copy(csr_hbm.at[pl.ds(mid, 16)],
                                scan_v.at[pl.ds(0, 16)])
                cmid = scan_v[pl.ds(0, 16)][0]
                small = hi - lo <= 64
                ge = cmid >= t
                new_lo = jnp.where(small, lo, jnp.where(ge, lo, mid + 1))
                new_hi = jnp.where(small, hi, jnp.where(ge, mid, hi))
                return (new_lo, new_hi)

            lo, hi = lax.fori_loop(0, 24, body, (jnp.int32(0),
                                                 jnp.int32(n_edges)))
            base = jnp.minimum(lo // 16 * 16, n_edges - 96)
            pltpu.sync_copy(csr_hbm.at[pl.ds(base, 96)], scan_v)

            cnt = jnp.int32(0)
            for k in range(6):
                v = scan_v[pl.ds(k * 16, 16)]
                for i in range(16):
                    cnt = cnt + jnp.where(v[i] < t, 1, 0)
            return base + cnt

        e_lo = lower_bound(wbase)
        e_hi = lower_bound(wbase + seg_per_w)

        a_lo = e_lo // 8 * 8
        nfull = (e_hi - a_lo) // chunk
        tail_proc = a_lo + nfull * chunk
        tail_base = jnp.minimum(tail_proc, n_edges - chunk)

        def start_load(base, par):
            boff = par * chunk
            pltpu.async_copy(ptrs_hbm.at[pl.ds(base, chunk)],
                             ptrs_v.at[pl.ds(boff, chunk)], lsem.at[par])
            pltpu.async_copy(csr_hbm.at[pl.ds(base, chunk)],
                             csr_v.at[pl.ds(boff, chunk)], lsem.at[par])

        def wait_load(base, par):
            boff = par * chunk
            pltpu.make_async_copy(ptrs_hbm.at[pl.ds(base, chunk)],
                                  ptrs_v.at[pl.ds(boff, chunk)],
                                  lsem.at[par]).wait()
            pltpu.make_async_copy(csr_hbm.at[pl.ds(base, chunk)],
                                  csr_v.at[pl.ds(boff, chunk)],
                                  lsem.at[par]).wait()

        def compute(base, par, lob, hib):
            boff = par * chunk

            @plsc.parallel_loop(0, vpc, unroll=12)
            def vbody(j):
                off = boff + j * 16
                seg = csr_v[pl.ds(off, 16)]
                segn = csr_v[pl.ds(off + 1, 16)]
                idx = ptrs_v[pl.ds(off, 16)]
                vals = plsc.load_gather(x_v, [idx])
                glob = (base + j * 16) + iota16
                inr = (glob >= lob) & (glob < hib)
                vals = jnp.where(inr, vals, jnp.float32(0.0))
                p = plsc.cumsum(vals)
                is_last = (seg != segn) | (iota16 == 15)
                offs = jnp.clip(seg - wbase, 0, win - 1)
                offsn = jnp.clip(segn - wbase, 0, win - 1)
                maska = is_last & inr
                inrn = (glob >= lob - 1) & (glob < hib - 1)
                maskb = is_last & inrn & not_last_lane
                plsc.addupdate_scatter(win_v, [offs], p, mask=maska)
                plsc.addupdate_scatter(win_v, [offsn], -p, mask=maskb)

        first_base = jnp.where(nfull > 0, a_lo, tail_base)
        start_load(first_base, 0)

        def mbody(i, carry):
            par = lax.rem(i, 2)
            base = a_lo + i * chunk
            wait_load(base, par)
            nbase = jnp.where(i + 1 < nfull, a_lo + (i + 1) * chunk,
                              tail_base)
            start_load(nbase, 1 - par)
            compute(base, par, e_lo, e_hi)
            return carry

        lax.fori_loop(0, nfull, mbody, 0)

        par_t = lax.rem(nfull, 2)
        wait_load(tail_base, par_t)
        compute(tail_base, par_t, jnp.maximum(tail_proc, e_lo), e_hi)

        pltpu.sync_copy(win_v.at[pl.ds(0, seg_per_w)],
                        out_hbm.at[pl.ds(wbase, seg_per_w)])

    return run


def kernel(x, ptrs, csr):
    n_edges = ptrs.shape[0]
    info = plsc.get_sparse_core_info()
    num_cores, num_subcores = info.num_cores, info.num_subcores
    n_workers = num_cores * num_subcores

    align = n_workers * 8
    n_out_pad = ((_N_SEG + align - 1) // align) * align
    chunk = 4992
    win = 3136

    run = _make_sc_kernel(n_out_pad, n_edges, chunk, win,
                          num_cores, num_subcores)
    out = run(x, ptrs, csr)
    return out[:_N_SEG]
